# baseline (speedup 1.0000x reference)
import functools

import jax
import jax.numpy as jnp
from jax import lax
from jax.experimental import pallas as pl
from jax.experimental.pallas import tpu as pltpu
from jax.experimental.pallas import tpu_sc as plsc

N_USERS = 10000
N_ITEMS = 40000
N = N_USERS + N_ITEMS
E = 800000
D = 64
D2 = D // 2

NUM_TILES = 16
BLK = 128
SB = 128
NSB = 400
CHUNK_SBS = 10
CHUNK = SB * CHUNK_SBS
NCHUNK = NSB // CHUNK_SBS
PE = NUM_TILES * NSB * SB
EPT = NSB * SB

ACC_ROWS = 50048
WR = 3128
WR_LAST = N - 15 * WR

_MESH = plsc.VectorSubcoreMesh(core_axis_name="c", subcore_axis_name="s")

_GATHER_DNUMS = lax.GatherDimensionNumbers(
    offset_dims=(), collapsed_slice_dims=(0,), start_index_map=(0,)
)


def _bcast_lane(vec16, e):
  idx = jnp.full((16, 1), e, dtype=jnp.int32)
  return lax.gather(
      vec16, idx, _GATHER_DNUMS, slice_sizes=(1,),
      mode=lax.GatherScatterMode.PROMISE_IN_BOUNDS,
  )


def _propagate_layer(x2, src, dst, w):
  @functools.partial(
      pl.kernel,
      out_type=jax.ShapeDtypeStruct((2, N, D2), jnp.float32),
      mesh=_MESH,
      compiler_params=pltpu.CompilerParams(use_tc_tiling_on_sc=False),
      scratch_types=[
          pltpu.VMEM_SHARED((ACC_ROWS, D2), jnp.float32),
          pltpu.VMEM((2, CHUNK), jnp.int32),
          pltpu.VMEM((2, CHUNK), jnp.int32),
          pltpu.VMEM((2, CHUNK), jnp.float32),
          pltpu.VMEM((2, 1, BLK), jnp.int32),
          pltpu.VMEM((2, SB, D2), jnp.float32),
          pltpu.SemaphoreType.DMA,
          pltpu.SemaphoreType.DMA,
          pltpu.SemaphoreType.DMA,
          pltpu.SemaphoreType.DMA,
          pltpu.SemaphoreType.DMA,
      ],
  )
  def layer_kernel(x_hbm, src_hbm, dst_hbm, w_hbm, y_hbm,
                   acc, sidx, didx, widx, adj, rows,
                   sem_idx, sem_g0, sem_g1, sem_s0, sem_s1):
    c = lax.axis_index("c")
    t = lax.axis_index("s")
    sem_g = (sem_g0, sem_g1)
    sem_s = (sem_s0, sem_s1)
    xc = x_hbm.at[c]

    zero16 = jnp.zeros((16,), jnp.float32)

    @pl.loop(0, SB)
    def _(r):
      for pp in range(2):
        for dd in range(D2 // 16):
          rows[pp, r, pl.ds(dd * 16, 16)] = zero16

    zb = t * WR
    for i in range(WR // SB):
      pltpu.sync_copy(rows.at[0], acc.at[pl.ds(zb + i * SB, SB)])
    pltpu.sync_copy(
        rows.at[0, pl.ds(0, WR - (WR // SB) * SB)],
        acc.at[pl.ds(zb + (WR // SB) * SB, WR - (WR // SB) * SB)])
    plsc.subcore_barrier()

    ebase = t * EPT

    def fire_idx_chunk(cn, qn):
      base = ebase + cn * CHUNK
      pltpu.async_copy(src_hbm.at[pl.ds(base, CHUNK)], sidx.at[qn], sem_idx)
      pltpu.async_copy(dst_hbm.at[pl.ds(base, CHUNK)], didx.at[qn], sem_idx)
      pltpu.async_copy(w_hbm.at[pl.ds(base, CHUNK)], widx.at[qn], sem_idx)

    def wait_idx_chunk(cn, qn):
      base = ebase + cn * CHUNK
      pltpu.make_async_copy(
          src_hbm.at[pl.ds(base, CHUNK)], sidx.at[qn], sem_idx).wait()
      pltpu.make_async_copy(
          dst_hbm.at[pl.ds(base, CHUNK)], didx.at[qn], sem_idx).wait()
      pltpu.make_async_copy(
          w_hbm.at[pl.ds(base, CHUNK)], widx.at[qn], sem_idx).wait()

    def fire_gather(soff, qn, pn):
      for j in range(SB // BLK):
        pltpu.async_copy(
            xc.at[sidx.at[qn, pl.ds(soff + j * BLK, BLK)]],
            rows.at[pn, pl.ds(j * BLK, BLK)], sem_g[pn])

    def wait_gather(soff, qn, pn):
      for j in range(SB // BLK):
        pltpu.make_async_copy(
            xc.at[sidx.at[qn, pl.ds(soff + j * BLK, BLK)]],
            rows.at[pn, pl.ds(j * BLK, BLK)], sem_g[pn]).wait()

    def fire_scatter(pn):
      for j in range(SB // BLK):
        pltpu.async_copy(
            rows.at[pn, pl.ds(j * BLK, BLK)],
            acc.at[adj.at[pn, j]], sem_s[pn], add=True)

    def wait_scatter(pn):
      for j in range(SB // BLK):
        pltpu.make_async_copy(
            rows.at[pn, pl.ds(j * BLK, BLK)],
            acc.at[adj.at[pn, j]], sem_s[pn]).wait()

    def compute_sb(soff, qn, pn):
      @plsc.parallel_loop(0, SB // 16, unroll=2)
      def _(k):
        adj[pn, 0, pl.ds(k * 16, 16)] = didx[qn, pl.ds(soff + k * 16, 16)]
        w_c = widx[qn, pl.ds(soff + k * 16, 16)]
        for g in range(2):
          wbs = [_bcast_lane(w_c, g * 8 + u) for u in range(8)]
          eis = [k * 16 + g * 8 + u for u in range(8)]
          vals = [
              [rows[pn, eis[u], pl.ds(dd * 16, 16)] for dd in range(D2 // 16)]
              for u in range(8)
          ]
          for u in range(8):
            for dd in range(D2 // 16):
              rows[pn, eis[u], pl.ds(dd * 16, 16)] = vals[u][dd] * wbs[u]

    fire_idx_chunk(0, 0)
    wait_idx_chunk(0, 0)
    fire_gather(0, 0, 0)

    @pl.loop(0, NCHUNK, step=2)
    def _(cc):
      for hq in range(2):
        cidx = cc + hq
        q = hq

        @pl.when(cidx < NCHUNK - 1)
        def _():
          fire_idx_chunk(cidx + 1, 1 - q)

        @pl.loop(0, CHUNK_SBS, step=2)
        def _(ss):
          for hp in range(2):
            s = ss + hp
            p = hp

            if hp == 0:
              @pl.when((cidx > 0) | (ss > 0))
              def _():
                wait_scatter(1 - p)
            else:
              wait_scatter(1 - p)

            if hp == 0:
              fire_gather((s + 1) * SB, q, 1 - p)
            else:
              @pl.when(ss < CHUNK_SBS - 2)
              def _():
                fire_gather((s + 1) * SB, q, 1 - p)

              @pl.when((ss == CHUNK_SBS - 2) & (cidx < NCHUNK - 1))
              def _():
                wait_idx_chunk(cidx + 1, 1 - q)
                fire_gather(0, 1 - q, 1 - p)

            wait_gather(s * SB, q, p)
            compute_sb(s * SB, q, p)
            fire_scatter(p)

    wait_scatter(1)
    plsc.subcore_barrier()

    @pl.when(t < 15)
    def _():
      pltpu.sync_copy(acc.at[pl.ds(t * WR, WR)], y_hbm.at[c, pl.ds(t * WR, WR)])

    @pl.when(t == 15)
    def _():
      pltpu.sync_copy(
          acc.at[pl.ds(15 * WR, WR_LAST)],
          y_hbm.at[c, pl.ds(15 * WR, WR_LAST)])

  return layer_kernel(x2, src, dst, w)


def _mean4(a, b, c, d):
  rows = 1000

  def body(a_ref, b_ref, c_ref, d_ref, o_ref):
    s = (a_ref[...] + b_ref[...] + c_ref[...] + d_ref[...]) * 0.25
    o_ref[...] = jnp.concatenate([s[0], s[1]], axis=-1)

  in_spec = pl.BlockSpec((2, rows, D2), lambda i: (0, i, 0))
  out_spec = pl.BlockSpec((rows, D), lambda i: (i, 0))
  return pl.pallas_call(
      body,
      out_shape=jax.ShapeDtypeStruct((N, D), jnp.float32),
      grid=(N // rows,),
      in_specs=[in_spec] * 4,
      out_specs=out_spec,
  )(a, b, c, d)


def kernel(user_emb, item_emb, edge_index, edge_weight):
  ego = jnp.concatenate([user_emb, item_emb], axis=0)
  ego2 = jnp.stack([ego[:, :D2], ego[:, D2:]], axis=0)
  pad = PE - E
  src = jnp.concatenate([edge_index[0], jnp.zeros((pad,), jnp.int32)])
  dst = jnp.concatenate([edge_index[1], jnp.zeros((pad,), jnp.int32)])
  w = jnp.concatenate([edge_weight, jnp.zeros((pad,), jnp.float32)])

  x1 = _propagate_layer(ego2, src, dst, w)
  xx2 = _propagate_layer(x1, src, dst, w)
  x3 = _propagate_layer(xx2, src, dst, w)

  m = _mean4(ego2, x1, xx2, x3)
  return m[:N_USERS], m[N_USERS:]


# speedup vs baseline: 1.2745x; 1.2745x over previous
import functools

import jax
import jax.numpy as jnp
from jax import lax
from jax.experimental import pallas as pl
from jax.experimental.pallas import tpu as pltpu
from jax.experimental.pallas import tpu_sc as plsc

N_USERS = 10000
N_ITEMS = 40000
N = N_USERS + N_ITEMS
E = 800000
D = 64
D2 = D // 2

NUM_TILES = 16
BLK = 128
SB = 128
NSB = 400
CHUNK_SBS = 10
CHUNK = SB * CHUNK_SBS
NCHUNK = NSB // CHUNK_SBS
PE = NUM_TILES * NSB * SB
EPT = NSB * SB

ACC_ROWS = 50048
WR = 3128
WR_LAST = N - 15 * WR

_MESH = plsc.VectorSubcoreMesh(core_axis_name="c", subcore_axis_name="s")

_GATHER_DNUMS = lax.GatherDimensionNumbers(
    offset_dims=(), collapsed_slice_dims=(0,), start_index_map=(0,)
)


def _bcast_lane(vec16, e):
  idx = jnp.full((16, 1), e, dtype=jnp.int32)
  return lax.gather(
      vec16, idx, _GATHER_DNUMS, slice_sizes=(1,),
      mode=lax.GatherScatterMode.PROMISE_IN_BOUNDS,
  )


def _propagate_layer(xb, src, dst, w):
  @functools.partial(
      pl.kernel,
      out_type=(jax.ShapeDtypeStruct((2, N, D2), jnp.float32),
                jax.ShapeDtypeStruct((2, N, D2), jnp.bfloat16)),
      mesh=_MESH,
      compiler_params=pltpu.CompilerParams(
          use_tc_tiling_on_sc=False, needs_layout_passes=False),
      scratch_types=[
          pltpu.VMEM_SHARED((ACC_ROWS, D2), jnp.float32),
          pltpu.VMEM((2, CHUNK), jnp.int32),
          pltpu.VMEM((2, CHUNK), jnp.int32),
          pltpu.VMEM((2, CHUNK), jnp.float32),
          pltpu.VMEM((2, 1, BLK), jnp.int32),
          pltpu.VMEM((2, SB, D2), jnp.bfloat16),
          pltpu.VMEM((2, SB, D2), jnp.float32),
          pltpu.SemaphoreType.DMA,
          pltpu.SemaphoreType.DMA,
          pltpu.SemaphoreType.DMA,
          pltpu.SemaphoreType.DMA,
          pltpu.SemaphoreType.DMA,
      ],
  )
  def layer_kernel(x_hbm, src_hbm, dst_hbm, w_hbm, y_hbm, yb_hbm,
                   acc, sidx, didx, widx, adj, grows, srows,
                   sem_idx, sem_g0, sem_g1, sem_s0, sem_s1):
    c = lax.axis_index("c")
    t = lax.axis_index("s")
    sem_g = (sem_g0, sem_g1)
    sem_s = (sem_s0, sem_s1)
    xc = x_hbm.at[c]

    zero16 = jnp.zeros((16,), jnp.float32)

    @pl.loop(0, SB)
    def _(r):
      for pp in range(2):
        for dd in range(D2 // 16):
          srows[pp, r, pl.ds(dd * 16, 16)] = zero16

    zb = t * WR
    for i in range(WR // SB):
      pltpu.sync_copy(srows.at[0], acc.at[pl.ds(zb + i * SB, SB)])
    pltpu.sync_copy(
        srows.at[0, pl.ds(0, WR - (WR // SB) * SB)],
        acc.at[pl.ds(zb + (WR // SB) * SB, WR - (WR // SB) * SB)])
    plsc.subcore_barrier()

    ebase = t * EPT

    def fire_idx_chunk(cn, qn):
      base = ebase + cn * CHUNK
      pltpu.async_copy(src_hbm.at[pl.ds(base, CHUNK)], sidx.at[qn], sem_idx)
      pltpu.async_copy(dst_hbm.at[pl.ds(base, CHUNK)], didx.at[qn], sem_idx)
      pltpu.async_copy(w_hbm.at[pl.ds(base, CHUNK)], widx.at[qn], sem_idx)

    def wait_idx_chunk(cn, qn):
      base = ebase + cn * CHUNK
      pltpu.make_async_copy(
          src_hbm.at[pl.ds(base, CHUNK)], sidx.at[qn], sem_idx).wait()
      pltpu.make_async_copy(
          dst_hbm.at[pl.ds(base, CHUNK)], didx.at[qn], sem_idx).wait()
      pltpu.make_async_copy(
          w_hbm.at[pl.ds(base, CHUNK)], widx.at[qn], sem_idx).wait()

    def fire_gather(soff, qn, pn):
      for j in range(SB // BLK):
        pltpu.async_copy(
            xc.at[sidx.at[qn, pl.ds(soff + j * BLK, BLK)]],
            grows.at[pn, pl.ds(j * BLK, BLK)], sem_g[pn])

    def wait_gather(soff, qn, pn):
      for j in range(SB // BLK):
        pltpu.make_async_copy(
            xc.at[sidx.at[qn, pl.ds(soff + j * BLK, BLK)]],
            grows.at[pn, pl.ds(j * BLK, BLK)], sem_g[pn]).wait()

    def fire_scatter(pn):
      for j in range(SB // BLK):
        pltpu.async_copy(
            srows.at[pn, pl.ds(j * BLK, BLK)],
            acc.at[adj.at[pn, j]], sem_s[pn], add=True)

    def wait_scatter(pn):
      for j in range(SB // BLK):
        pltpu.make_async_copy(
            srows.at[pn, pl.ds(j * BLK, BLK)],
            acc.at[adj.at[pn, j]], sem_s[pn]).wait()

    def compute_sb(soff, qn, pn):
      @plsc.parallel_loop(0, SB // 16, unroll=2)
      def _(k):
        adj[pn, 0, pl.ds(k * 16, 16)] = didx[qn, pl.ds(soff + k * 16, 16)]
        w_c = widx[qn, pl.ds(soff + k * 16, 16)]
        for g in range(2):
          wbs = [_bcast_lane(w_c, g * 8 + u) for u in range(8)]
          eis = [k * 16 + g * 8 + u for u in range(8)]
          bvs = [grows[pn, eis[u], pl.ds(0, D2)] for u in range(8)]
          lohi = [plsc.unpack(bv, format=plsc.PackFormat.INTERLEAVED)
                  for bv in bvs]
          for u in range(8):
            srows[pn, eis[u], pl.ds(0, 16)] = lohi[u][0] * wbs[u]
            srows[pn, eis[u], pl.ds(16, 16)] = lohi[u][1] * wbs[u]

    fire_idx_chunk(0, 0)
    wait_idx_chunk(0, 0)
    fire_gather(0, 0, 0)

    @pl.loop(0, NCHUNK, step=2)
    def _(cc):
      for hq in range(2):
        cidx = cc + hq
        q = hq

        @pl.when(cidx < NCHUNK - 1)
        def _():
          fire_idx_chunk(cidx + 1, 1 - q)

        @pl.loop(0, CHUNK_SBS, step=2)
        def _(ss):
          for hp in range(2):
            s = ss + hp
            p = hp

            if hp == 0:
              @pl.when((cidx > 0) | (ss > 0))
              def _():
                wait_scatter(1 - p)
            else:
              wait_scatter(1 - p)

            if hp == 0:
              fire_gather((s + 1) * SB, q, 1 - p)
            else:
              @pl.when(ss < CHUNK_SBS - 2)
              def _():
                fire_gather((s + 1) * SB, q, 1 - p)

              @pl.when((ss == CHUNK_SBS - 2) & (cidx < NCHUNK - 1))
              def _():
                wait_idx_chunk(cidx + 1, 1 - q)
                fire_gather(0, 1 - q, 1 - p)

            wait_gather(s * SB, q, p)
            compute_sb(s * SB, q, p)
            fire_scatter(p)

    wait_scatter(1)
    plsc.subcore_barrier()

    def write_rows(r0, nrows):
      pltpu.sync_copy(acc.at[pl.ds(r0, nrows)], srows.at[0, pl.ds(0, nrows)])

      @pl.loop(0, nrows)
      def _(r):
        lo = srows[0, r, pl.ds(0, 16)]
        hi = srows[0, r, pl.ds(16, 16)]
        grows[0, r, pl.ds(0, D2)] = plsc.pack(
            lo, hi, format=plsc.PackFormat.INTERLEAVED)

      pltpu.sync_copy(srows.at[0, pl.ds(0, nrows)],
                      y_hbm.at[c, pl.ds(r0, nrows)])
      pltpu.sync_copy(grows.at[0, pl.ds(0, nrows)],
                      yb_hbm.at[c, pl.ds(r0, nrows)])

    for i in range(WR // SB):
      write_rows(zb + i * SB, SB)

    @pl.when(t < 15)
    def _():
      write_rows(zb + (WR // SB) * SB, WR - (WR // SB) * SB)

    @pl.when(t == 15)
    def _():
      write_rows(zb + (WR // SB) * SB, WR_LAST - (WR // SB) * SB)

  return layer_kernel(xb, src, dst, w)


TOT = 2 * N * D2
CPW = TOT // 32
CCH = 4000


def _to_bf16(flat):
  @functools.partial(
      pl.kernel,
      out_type=jax.ShapeDtypeStruct((TOT,), jnp.bfloat16),
      mesh=_MESH,
      compiler_params=pltpu.CompilerParams(
          use_tc_tiling_on_sc=False, needs_layout_passes=False),
      scratch_types=[
          pltpu.VMEM((CCH,), jnp.float32),
          pltpu.VMEM((CCH,), jnp.bfloat16),
      ],
  )
  def conv_kernel(x_hbm, o_hbm, fbuf, bbuf):
    wid = lax.axis_index("s") * 2 + lax.axis_index("c")
    base = wid * CPW

    @pl.loop(0, CPW // CCH)
    def _(i):
      b0 = base + i * CCH
      pltpu.sync_copy(x_hbm.at[pl.ds(b0, CCH)], fbuf)

      @pl.loop(0, CCH // 32)
      def _(kk):
        lo = fbuf[pl.ds(kk * 32, 16)]
        hi = fbuf[pl.ds(kk * 32 + 16, 16)]
        bbuf[pl.ds(kk * 32, 32)] = plsc.pack(
            lo, hi, format=plsc.PackFormat.INTERLEAVED)

      pltpu.sync_copy(bbuf, o_hbm.at[pl.ds(b0, CCH)])

  return conv_kernel(flat)


def _mean4(a, b, c, d):
  rows = 1000

  def body(a_ref, b_ref, c_ref, d_ref, o_ref):
    s = (a_ref[...] + b_ref[...] + c_ref[...] + d_ref[...]) * 0.25
    o_ref[...] = jnp.concatenate([s[0], s[1]], axis=-1)

  in_spec = pl.BlockSpec((2, rows, D2), lambda i: (0, i, 0))
  out_spec = pl.BlockSpec((rows, D), lambda i: (i, 0))
  return pl.pallas_call(
      body,
      out_shape=jax.ShapeDtypeStruct((N, D), jnp.float32),
      grid=(N // rows,),
      in_specs=[in_spec] * 4,
      out_specs=out_spec,
  )(a, b, c, d)


def kernel(user_emb, item_emb, edge_index, edge_weight):
  ego = jnp.concatenate([user_emb, item_emb], axis=0)
  ego2 = jnp.stack([ego[:, :D2], ego[:, D2:]], axis=0)
  pad = PE - E
  src = jnp.concatenate([edge_index[0], jnp.zeros((pad,), jnp.int32)])
  dst = jnp.concatenate([edge_index[1], jnp.zeros((pad,), jnp.int32)])
  w = jnp.concatenate([edge_weight, jnp.zeros((pad,), jnp.float32)])

  ego_b = _to_bf16(ego2.reshape(-1)).reshape(2, N, D2)
  x1, x1b = _propagate_layer(ego_b, src, dst, w)
  xx2, x2b = _propagate_layer(x1b, src, dst, w)
  x3, _ = _propagate_layer(x2b, src, dst, w)

  m = _mean4(ego2, x1, xx2, x3)
  return m[:N_USERS], m[N_USERS:]


# speedup vs baseline: 1.3324x; 1.0454x over previous
import functools

import jax
import jax.numpy as jnp
from jax import lax
from jax.experimental import pallas as pl
from jax.experimental.pallas import tpu as pltpu
from jax.experimental.pallas import tpu_sc as plsc

N_USERS = 10000
N_ITEMS = 40000
N = N_USERS + N_ITEMS
E = 800000
D = 64
D2 = D // 2

NUM_TILES = 16
BLK = 128
SB = 128
NSB = 400
CHUNK_SBS = 10
CHUNK = SB * CHUNK_SBS
NCHUNK = NSB // CHUNK_SBS
PE = NUM_TILES * NSB * SB
EPT = NSB * SB

ACC_ROWS = 50048
WR = 3128
WR_LAST = N - 15 * WR

_MESH = plsc.VectorSubcoreMesh(core_axis_name="c", subcore_axis_name="s")

_GATHER_DNUMS = lax.GatherDimensionNumbers(
    offset_dims=(), collapsed_slice_dims=(0,), start_index_map=(0,)
)


def _bcast_lane(vec16, e):
  idx = jnp.full((16, 1), e, dtype=jnp.int32)
  return lax.gather(
      vec16, idx, _GATHER_DNUMS, slice_sizes=(1,),
      mode=lax.GatherScatterMode.PROMISE_IN_BOUNDS,
  )


def _propagate_layer(x_in, src, dst, w, first=False, last=False):
  outs = [jax.ShapeDtypeStruct((2, N, D2), jnp.float32)]
  if not last:
    outs.append(jax.ShapeDtypeStruct((2, N, D2), jnp.bfloat16))
  if first:
    outs.append(jax.ShapeDtypeStruct((2, N, D2), jnp.bfloat16))

  @functools.partial(
      pl.kernel,
      out_type=tuple(outs),
      mesh=_MESH,
      compiler_params=pltpu.CompilerParams(
          use_tc_tiling_on_sc=False, needs_layout_passes=False),
      scratch_types=[
          pltpu.VMEM_SHARED((ACC_ROWS, D2), jnp.float32),
          pltpu.VMEM((2, CHUNK), jnp.int32),
          pltpu.VMEM((2, CHUNK), jnp.int32),
          pltpu.VMEM((2, CHUNK), jnp.float32),
          pltpu.VMEM((2, 1, BLK), jnp.int32),
          pltpu.VMEM((2, SB, D2), jnp.bfloat16),
          pltpu.VMEM((2, SB, D2), jnp.float32),
          pltpu.SemaphoreType.DMA,
          pltpu.SemaphoreType.DMA,
          pltpu.SemaphoreType.DMA,
          pltpu.SemaphoreType.DMA,
          pltpu.SemaphoreType.DMA,
      ],
  )
  def layer_kernel(*refs):
    rl = list(refs)
    x_hbm, src_hbm, dst_hbm, w_hbm, y_hbm = rl[:5]
    pos = 5
    yb_hbm = None
    egob_hbm = None
    if not last:
      yb_hbm = rl[pos]
      pos += 1
    if first:
      egob_hbm = rl[pos]
      pos += 1
    (acc, sidx, didx, widx, adj, grows, srows,
     sem_idx, sem_g0, sem_g1, sem_s0, sem_s1) = rl[pos:]
    c = lax.axis_index("c")
    t = lax.axis_index("s")
    sem_g = (sem_g0, sem_g1)
    sem_s = (sem_s0, sem_s1)

    zb = t * WR
    tail0 = (WR // SB) * SB
    ntail = WR - tail0
    ntail_l = WR_LAST - tail0

    def stream_pack(read_at, write_at):
      nch = WR // SB

      def pack_chunk(p, nrows):
        @pl.loop(0, nrows)
        def _(r):
          lo = srows[p, r, pl.ds(0, 16)]
          hi = srows[p, r, pl.ds(16, 16)]
          grows[p, r, pl.ds(0, D2)] = plsc.pack(
              lo, hi, format=plsc.PackFormat.INTERLEAVED)

      pltpu.async_copy(read_at(zb, SB), srows.at[0], sem_g[0])
      for i in range(nch):
        p = i & 1
        pltpu.make_async_copy(
            read_at(zb + i * SB, SB), srows.at[p], sem_g[p]).wait()
        if i + 1 < nch:
          pltpu.async_copy(
              read_at(zb + (i + 1) * SB, SB), srows.at[1 - p], sem_g[1 - p])
        if i >= 2:
          pltpu.make_async_copy(
              grows.at[p], write_at(zb + (i - 2) * SB, SB), sem_s[p]).wait()
        pack_chunk(p, SB)
        pltpu.async_copy(grows.at[p], write_at(zb + i * SB, SB), sem_s[p])
      for i in (nch - 2, nch - 1):
        p = i & 1
        pltpu.make_async_copy(
            grows.at[p], write_at(zb + i * SB, SB), sem_s[p]).wait()

      def do_tail(n):
        pltpu.sync_copy(read_at(zb + tail0, n), srows.at[0, pl.ds(0, n)])
        pack_chunk(0, n)
        pltpu.sync_copy(grows.at[0, pl.ds(0, n)], write_at(zb + tail0, n))

      @pl.when(t < 15)
      def _():
        do_tail(ntail)

      @pl.when(t == 15)
      def _():
        do_tail(ntail_l)

    if first:
      stream_pack(lambda r0, n: x_hbm.at[c, pl.ds(r0, n)],
                  lambda r0, n: egob_hbm.at[c, pl.ds(r0, n)])
      xc = egob_hbm.at[c]
    else:
      xc = x_hbm.at[c]

    zero16 = jnp.zeros((16,), jnp.float32)

    @pl.loop(0, SB)
    def _(r):
      for pp in range(2):
        for dd in range(D2 // 16):
          srows[pp, r, pl.ds(dd * 16, 16)] = zero16

    for i in range(WR // SB):
      pltpu.async_copy(srows.at[0], acc.at[pl.ds(zb + i * SB, SB)], sem_idx)
    pltpu.async_copy(srows.at[0, pl.ds(0, ntail)],
                     acc.at[pl.ds(zb + tail0, ntail)], sem_idx)
    for i in range(WR // SB):
      pltpu.make_async_copy(
          srows.at[0], acc.at[pl.ds(zb + i * SB, SB)], sem_idx).wait()
    pltpu.make_async_copy(srows.at[0, pl.ds(0, ntail)],
                          acc.at[pl.ds(zb + tail0, ntail)], sem_idx).wait()
    plsc.subcore_barrier()

    ebase = t * EPT

    def fire_idx_chunk(cn, qn):
      base = ebase + cn * CHUNK
      pltpu.async_copy(src_hbm.at[pl.ds(base, CHUNK)], sidx.at[qn], sem_idx)
      pltpu.async_copy(dst_hbm.at[pl.ds(base, CHUNK)], didx.at[qn], sem_idx)
      pltpu.async_copy(w_hbm.at[pl.ds(base, CHUNK)], widx.at[qn], sem_idx)

    def wait_idx_chunk(cn, qn):
      base = ebase + cn * CHUNK
      pltpu.make_async_copy(
          src_hbm.at[pl.ds(base, CHUNK)], sidx.at[qn], sem_idx).wait()
      pltpu.make_async_copy(
          dst_hbm.at[pl.ds(base, CHUNK)], didx.at[qn], sem_idx).wait()
      pltpu.make_async_copy(
          w_hbm.at[pl.ds(base, CHUNK)], widx.at[qn], sem_idx).wait()

    def fire_gather(soff, qn, pn):
      for j in range(SB // BLK):
        pltpu.async_copy(
            xc.at[sidx.at[qn, pl.ds(soff + j * BLK, BLK)]],
            grows.at[pn, pl.ds(j * BLK, BLK)], sem_g[pn])

    def wait_gather(soff, qn, pn):
      for j in range(SB // BLK):
        pltpu.make_async_copy(
            xc.at[sidx.at[qn, pl.ds(soff + j * BLK, BLK)]],
            grows.at[pn, pl.ds(j * BLK, BLK)], sem_g[pn]).wait()

    def fire_scatter(pn):
      for j in range(SB // BLK):
        pltpu.async_copy(
            srows.at[pn, pl.ds(j * BLK, BLK)],
            acc.at[adj.at[pn, j]], sem_s[pn], add=True)

    def wait_scatter(pn):
      for j in range(SB // BLK):
        pltpu.make_async_copy(
            srows.at[pn, pl.ds(j * BLK, BLK)],
            acc.at[adj.at[pn, j]], sem_s[pn]).wait()

    def compute_sb(soff, qn, pn):
      @plsc.parallel_loop(0, SB // 16, unroll=2)
      def _(k):
        adj[pn, 0, pl.ds(k * 16, 16)] = didx[qn, pl.ds(soff + k * 16, 16)]
        w_c = widx[qn, pl.ds(soff + k * 16, 16)]
        for g in range(2):
          wbs = [_bcast_lane(w_c, g * 8 + u) for u in range(8)]
          eis = [k * 16 + g * 8 + u for u in range(8)]
          bvs = [grows[pn, eis[u], pl.ds(0, D2)] for u in range(8)]
          lohi = [plsc.unpack(bv, format=plsc.PackFormat.INTERLEAVED)
                  for bv in bvs]
          for u in range(8):
            srows[pn, eis[u], pl.ds(0, 16)] = lohi[u][0] * wbs[u]
            srows[pn, eis[u], pl.ds(16, 16)] = lohi[u][1] * wbs[u]

    fire_idx_chunk(0, 0)
    wait_idx_chunk(0, 0)
    fire_gather(0, 0, 0)

    @pl.loop(0, NCHUNK, step=2)
    def _(cc):
      for hq in range(2):
        cidx = cc + hq
        q = hq

        @pl.when(cidx < NCHUNK - 1)
        def _():
          fire_idx_chunk(cidx + 1, 1 - q)

        @pl.loop(0, CHUNK_SBS, step=2)
        def _(ss):
          for hp in range(2):
            s = ss + hp
            p = hp

            if hp == 0:
              @pl.when((cidx > 0) | (ss > 0))
              def _():
                wait_scatter(1 - p)
            else:
              wait_scatter(1 - p)

            if hp == 0:
              fire_gather((s + 1) * SB, q, 1 - p)
            else:
              @pl.when(ss < CHUNK_SBS - 2)
              def _():
                fire_gather((s + 1) * SB, q, 1 - p)

              @pl.when((ss == CHUNK_SBS - 2) & (cidx < NCHUNK - 1))
              def _():
                wait_idx_chunk(cidx + 1, 1 - q)
                fire_gather(0, 1 - q, 1 - p)

            wait_gather(s * SB, q, p)
            compute_sb(s * SB, q, p)
            fire_scatter(p)

    wait_scatter(1)
    plsc.subcore_barrier()

    @pl.when(t < 15)
    def _():
      pltpu.async_copy(acc.at[pl.ds(zb, WR)],
                       y_hbm.at[c, pl.ds(zb, WR)], sem_idx)

    @pl.when(t == 15)
    def _():
      pltpu.async_copy(acc.at[pl.ds(zb, WR_LAST)],
                       y_hbm.at[c, pl.ds(zb, WR_LAST)], sem_idx)

    if not last:
      stream_pack(lambda r0, n: acc.at[pl.ds(r0, n)],
                  lambda r0, n: yb_hbm.at[c, pl.ds(r0, n)])

    @pl.when(t < 15)
    def _():
      pltpu.make_async_copy(acc.at[pl.ds(zb, WR)],
                            y_hbm.at[c, pl.ds(zb, WR)], sem_idx).wait()

    @pl.when(t == 15)
    def _():
      pltpu.make_async_copy(acc.at[pl.ds(zb, WR_LAST)],
                            y_hbm.at[c, pl.ds(zb, WR_LAST)], sem_idx).wait()

  res = layer_kernel(x_in, src, dst, w)
  if last:
    return res[0], None
  return res[0], res[1]


def _mean4(a, b, c, d):
  rows = 1000

  def body(a_ref, b_ref, c_ref, d_ref, o_ref):
    s = (a_ref[...] + b_ref[...] + c_ref[...] + d_ref[...]) * 0.25
    o_ref[...] = jnp.concatenate([s[0], s[1]], axis=-1)

  in_spec = pl.BlockSpec((2, rows, D2), lambda i: (0, i, 0))
  out_spec = pl.BlockSpec((rows, D), lambda i: (i, 0))
  return pl.pallas_call(
      body,
      out_shape=jax.ShapeDtypeStruct((N, D), jnp.float32),
      grid=(N // rows,),
      in_specs=[in_spec] * 4,
      out_specs=out_spec,
  )(a, b, c, d)


def kernel(user_emb, item_emb, edge_index, edge_weight):
  ego = jnp.concatenate([user_emb, item_emb], axis=0)
  ego2 = jnp.stack([ego[:, :D2], ego[:, D2:]], axis=0)
  pad = PE - E
  src = jnp.concatenate([edge_index[0], jnp.zeros((pad,), jnp.int32)])
  dst = jnp.concatenate([edge_index[1], jnp.zeros((pad,), jnp.int32)])
  w = jnp.concatenate([edge_weight, jnp.zeros((pad,), jnp.float32)])

  x1, x1b = _propagate_layer(ego2, src, dst, w, first=True)
  xx2, x2b = _propagate_layer(x1b, src, dst, w)
  x3, _ = _propagate_layer(x2b, src, dst, w, last=True)

  m = _mean4(ego2, x1, xx2, x3)
  return m[:N_USERS], m[N_USERS:]
